# Initial kernel scaffold; baseline (speedup 1.0000x reference)
import jax
import jax.numpy as jnp
from jax.experimental import pallas as pl


def kernel(x, edge_index, W1, b1, W2, b2):
    raise NotImplementedError("write your pallas kernel here")


# speedup vs baseline: 7.1214x; 7.1214x over previous
import functools

import jax
import jax.numpy as jnp
from jax.experimental import pallas as pl
from jax.experimental.pallas import tpu as pltpu
from jax.experimental.pallas import tpu_sc as plsc

ALPHA = 0.1
K = 10
NS = 16
EB = 128
RSUB = 128
NPAD = 10240


def _sc_hist(dst, n, e):
    ept = e // NS
    nfull = ept // EB
    erem = ept % EB
    rpt = n // NS
    mesh = plsc.VectorSubcoreMesh(core_axis_name="c", subcore_axis_name="s")

    @functools.partial(
        pl.kernel,
        out_type=jax.ShapeDtypeStruct((n, 16), jnp.float32),
        mesh=mesh,
        compiler_params=pltpu.CompilerParams(use_tc_tiling_on_sc=False),
        scratch_types=[
            pltpu.VMEM_SHARED((n, 16), jnp.float32),
            pltpu.VMEM((rpt, 16), jnp.float32),
            pltpu.VMEM((EB, 16), jnp.float32),
            pltpu.VMEM((EB,), jnp.int32),
            pltpu.VMEM((erem, 16), jnp.float32),
            pltpu.VMEM((erem,), jnp.int32),
        ],
    )
    def hist_kernel(dst_hbm, out_hbm, acc, rowbuf, ones_b, didx, ones_r, didx_r):
        core = jax.lax.axis_index("c")
        tile = jax.lax.axis_index("s")

        @pl.when(core == 0)
        def _():
            zeros16 = jnp.zeros((16,), jnp.float32)
            ones16 = jnp.ones((16,), jnp.float32)

            @pl.loop(0, rpt)
            def _(j):
                rowbuf[j, :] = zeros16

            @pl.loop(0, EB)
            def _(j):
                ones_b[j, :] = ones16

            @pl.loop(0, erem)
            def _(j):
                ones_r[j, :] = ones16

            pltpu.sync_copy(rowbuf, acc.at[pl.ds(tile * rpt, rpt)])
            plsc.subcore_barrier()

            @pl.loop(0, nfull)
            def _(i):
                off = tile * ept + i * EB
                pltpu.sync_copy(dst_hbm.at[pl.ds(off, EB)], didx)
                pltpu.sync_copy(ones_b, acc.at[didx], add=True)

            off = tile * ept + nfull * EB
            pltpu.sync_copy(dst_hbm.at[pl.ds(off, erem)], didx_r)
            pltpu.sync_copy(ones_r, acc.at[didx_r], add=True)

            plsc.subcore_barrier()
            pltpu.sync_copy(acc.at[pl.ds(tile * rpt, rpt)], rowbuf)
            pltpu.sync_copy(rowbuf, out_hbm.at[pl.ds(tile * rpt, rpt)])

    return hist_kernel(dst)


def _tc_prep(x, W1, b1, W2, b2, deg16):
    n = x.shape[0]
    c = W2.shape[1]
    fo = jax.ShapeDtypeStruct((n, c), jnp.float32)

    def body(x_ref, w1_ref, b1_ref, w2_ref, b2_ref, deg_ref,
             zt0_ref, u_ref, ht_ref, din_ref, ah_ref):
        h1 = jnp.maximum(
            jnp.dot(x_ref[...], w1_ref[...], preferred_element_type=jnp.float32)
            + b1_ref[...], 0.0)
        h = jnp.dot(h1, w2_ref[...], preferred_element_type=jnp.float32) + b2_ref[...]
        deg = deg_ref[...][:, 0:1] + 1.0
        dinv = jax.lax.rsqrt(deg)
        zt0_ref[...] = dinv * h
        u_ref[...] = jnp.broadcast_to((1.0 - ALPHA) * dinv * dinv, h.shape)
        ht_ref[...] = ALPHA * dinv * h
        din_ref[...] = jnp.broadcast_to((1.0 - ALPHA) * dinv, h.shape)
        ah_ref[...] = ALPHA * h

    return pl.pallas_call(
        body, out_shape=(fo, fo, fo, fo, fo),
    )(x, W1, b1.reshape(1, -1), W2, b2.reshape(1, -1), deg16)


def _sc_prop(zt0, src, dst, u64, ht64):
    n, c = zt0.shape
    e = src.shape[0]
    ept = e // NS
    nfull = ept // EB
    erem = ept % EB
    rpt = n // NS
    nsub = rpt // RSUB
    mesh = plsc.VectorSubcoreMesh(core_axis_name="c", subcore_axis_name="s")
    fo = jax.ShapeDtypeStruct((n, c), jnp.float32)

    @functools.partial(
        pl.kernel,
        out_type=(fo, fo),
        mesh=mesh,
        compiler_params=pltpu.CompilerParams(use_tc_tiling_on_sc=False),
        scratch_types=[
            pltpu.VMEM_SHARED((n, c), jnp.float32),
            pltpu.VMEM((EB, c), jnp.float32),
            pltpu.VMEM((EB,), jnp.int32),
            pltpu.VMEM((EB,), jnp.int32),
            pltpu.VMEM((erem, c), jnp.float32),
            pltpu.VMEM((erem,), jnp.int32),
            pltpu.VMEM((erem,), jnp.int32),
            pltpu.VMEM((RSUB, c), jnp.float32),
            pltpu.VMEM((RSUB, c), jnp.float32),
            pltpu.VMEM((RSUB, c), jnp.float32),
        ],
    )
    def prop_kernel(zt0_hbm, src_hbm, dst_hbm, u_hbm, ht_hbm,
                    sout_hbm, ztb_hbm, agg,
                    rows, sidx, didx, rows_r, sidx_r, didx_r,
                    abuf, ubuf, hbuf):
        core = jax.lax.axis_index("c")
        tile = jax.lax.axis_index("s")

        @pl.when(core == 0)
        def _():
            for sub in range(nsub):
                r0 = tile * rpt + sub * RSUB
                sl = pl.ds(r0, RSUB)
                pltpu.sync_copy(zt0_hbm.at[sl], abuf)
                pltpu.sync_copy(abuf, agg.at[sl])
                pltpu.sync_copy(abuf, ztb_hbm.at[sl])
            plsc.subcore_barrier()

            def edge_pass():
                @pl.loop(0, nfull)
                def _(i):
                    off = tile * ept + i * EB
                    pltpu.sync_copy(src_hbm.at[pl.ds(off, EB)], sidx)
                    pltpu.sync_copy(dst_hbm.at[pl.ds(off, EB)], didx)
                    pltpu.sync_copy(ztb_hbm.at[sidx], rows)
                    pltpu.sync_copy(rows, agg.at[didx], add=True)

                off = tile * ept + nfull * EB
                pltpu.sync_copy(src_hbm.at[pl.ds(off, erem)], sidx_r)
                pltpu.sync_copy(dst_hbm.at[pl.ds(off, erem)], didx_r)
                pltpu.sync_copy(ztb_hbm.at[sidx_r], rows_r)
                pltpu.sync_copy(rows_r, agg.at[didx_r], add=True)

            @pl.loop(0, K - 1)
            def _(k):
                edge_pass()
                plsc.subcore_barrier()
                for sub in range(nsub):
                    r0 = tile * rpt + sub * RSUB
                    sl = pl.ds(r0, RSUB)
                    pltpu.sync_copy(agg.at[sl], abuf)
                    pltpu.sync_copy(u_hbm.at[sl], ubuf)
                    pltpu.sync_copy(ht_hbm.at[sl], hbuf)

                    @pl.loop(0, RSUB)
                    def _(j):
                        for cc in range(c // 16):
                            csl = pl.ds(cc * 16, 16)
                            abuf[j, csl] = (ubuf[j, csl] * abuf[j, csl]
                                            + hbuf[j, csl])

                    pltpu.sync_copy(abuf, ztb_hbm.at[sl])
                    pltpu.sync_copy(abuf, agg.at[sl])
                plsc.subcore_barrier()

            edge_pass()
            plsc.subcore_barrier()
            for sub in range(nsub):
                r0 = tile * rpt + sub * RSUB
                sl = pl.ds(r0, RSUB)
                pltpu.sync_copy(agg.at[sl], abuf)
                pltpu.sync_copy(abuf, sout_hbm.at[sl])

    return prop_kernel(zt0, src, dst, u64, ht64)[0]


def _tc_finish(s, din64, ah64):
    n, c = s.shape

    def body(s_ref, din_ref, ah_ref, o_ref):
        z = din_ref[...] * s_ref[...] + ah_ref[...]
        m = jnp.max(z, axis=1, keepdims=True)
        lse = jnp.log(jnp.sum(jnp.exp(z - m), axis=1, keepdims=True)) + m
        o_ref[...] = z - lse

    return pl.pallas_call(
        body, out_shape=jax.ShapeDtypeStruct((n, c), jnp.float32),
    )(s, din64, ah64)


def kernel(x, edge_index, W1, b1, W2, b2):
    n = x.shape[0]
    e = edge_index.shape[1]
    src = edge_index[0]
    dst = edge_index[1]
    xp = jnp.pad(x, ((0, NPAD - n), (0, 0)))
    deg16 = _sc_hist(dst, NPAD, e)
    zt0, u64, ht64, din64, ah64 = _tc_prep(xp, W1, b1, W2, b2, deg16)
    s = _sc_prop(zt0, src, dst, u64, ht64)
    return _tc_finish(s, din64, ah64)[:n]


# speedup vs baseline: 22.9277x; 3.2196x over previous
import functools

import jax
import jax.numpy as jnp
from jax.experimental import pallas as pl
from jax.experimental.pallas import tpu as pltpu
from jax.experimental.pallas import tpu_sc as plsc

ALPHA = 0.1
K = 10
NS = 16
EB = 128
NCH = 160
RSUB = 128
NPAD = 10240
CH = 32

_SC_PARAMS = pltpu.CompilerParams(use_tc_tiling_on_sc=False)
_MESH = dict(core_axis_name="c", subcore_axis_name="s")


def _sc_hist(dst2d):
    hpt = NCH // 2
    rpt = NPAD // NS

    @functools.partial(
        pl.kernel,
        out_type=jax.ShapeDtypeStruct((2, NPAD, 16), jnp.float32),
        mesh=plsc.VectorSubcoreMesh(**_MESH),
        compiler_params=_SC_PARAMS,
        scratch_types=[
            pltpu.VMEM_SHARED((NPAD, 16), jnp.float32),
            pltpu.VMEM((rpt, 16), jnp.float32),
            pltpu.VMEM((EB, 16), jnp.float32),
            pltpu.VMEM((EB,), jnp.int32),
        ],
    )
    def hist_kernel(dst_hbm, out_hbm, acc, rowbuf, ones_b, didx):
        core = jax.lax.axis_index("c")
        tile = jax.lax.axis_index("s")
        zeros16 = jnp.zeros((16,), jnp.float32)
        ones16 = jnp.ones((16,), jnp.float32)

        @pl.loop(0, rpt)
        def _(j):
            rowbuf[j, :] = zeros16

        pltpu.sync_copy(rowbuf, acc.at[pl.ds(tile * rpt, rpt)])

        @pl.loop(0, EB)
        def _(j):
            ones_b[j, :] = ones16

        plsc.subcore_barrier()
        c0 = tile * NCH + core * hpt

        @pl.loop(0, hpt)
        def _(i):
            pltpu.sync_copy(dst_hbm.at[c0 + i], didx)
            pltpu.sync_copy(ones_b, acc.at[didx], add=True)

        plsc.subcore_barrier()
        pltpu.sync_copy(acc.at[pl.ds(tile * rpt, rpt)], rowbuf)
        pltpu.sync_copy(rowbuf, out_hbm.at[core].at[pl.ds(tile * rpt, rpt)])

    return hist_kernel(dst2d)


def _tc_prep(x, W1, b1, W2, b2, deg16):
    n = x.shape[0]
    c = W2.shape[1]

    blk = 1280
    grid = n // blk

    def body(x_ref, w1_ref, b1_ref, w2_ref, b2_ref, deg_ref,
             ztlo_ref, zthi_ref, u_ref, htlo_ref, hthi_ref, din_ref, ah_ref):
        h1 = jnp.maximum(
            jnp.dot(x_ref[...], w1_ref[...], preferred_element_type=jnp.float32)
            + b1_ref[...], 0.0)
        h = jnp.dot(h1, w2_ref[...], preferred_element_type=jnp.float32) + b2_ref[...]
        deg = deg_ref[...][0, :, 0:1] + deg_ref[...][1, :, 0:1] + 1.0
        dinv = jax.lax.rsqrt(deg)
        zt = dinv * h
        ztlo_ref[...] = zt[:, :CH]
        zthi_ref[...] = zt[:, CH:]
        htlo_ref[...] = ALPHA * zt[:, :CH]
        hthi_ref[...] = ALPHA * zt[:, CH:]
        u_ref[...] = jnp.broadcast_to((1.0 - ALPHA) * dinv * dinv, (blk, CH))
        din_ref[...] = jnp.broadcast_to((1.0 - ALPHA) * dinv, h.shape)
        ah_ref[...] = ALPHA * h

    f = jnp.float32
    row = lambda i: (i, 0)
    bs_h = pl.BlockSpec((blk, CH), row)
    bs_c = pl.BlockSpec((blk, c), row)
    return pl.pallas_call(
        body,
        grid=(grid,),
        in_specs=[
            pl.BlockSpec((blk, x.shape[1]), row),
            pl.BlockSpec(W1.shape, lambda i: (0, 0)),
            pl.BlockSpec((1, b1.shape[0]), lambda i: (0, 0)),
            pl.BlockSpec(W2.shape, lambda i: (0, 0)),
            pl.BlockSpec((1, b2.shape[0]), lambda i: (0, 0)),
            pl.BlockSpec((2, blk, 16), lambda i: (0, i, 0)),
        ],
        out_specs=[bs_h, bs_h, bs_h, bs_h, bs_h, bs_c, bs_c],
        out_shape=(
            jax.ShapeDtypeStruct((n, CH), f),
            jax.ShapeDtypeStruct((n, CH), f),
            jax.ShapeDtypeStruct((n, CH), f),
            jax.ShapeDtypeStruct((n, CH), f),
            jax.ShapeDtypeStruct((n, CH), f),
            jax.ShapeDtypeStruct((n, c), f),
            jax.ShapeDtypeStruct((n, c), f),
        ),
    )(x, W1, b1.reshape(1, -1), W2, b2.reshape(1, -1), deg16)


def _sc_prop(zt0f, src2d, dst2d, u32, htf):
    rpt = NPAD // NS
    nsub = rpt // RSUB

    @functools.partial(
        pl.kernel,
        out_type=jax.ShapeDtypeStruct((2 * NPAD, CH), jnp.float32),
        mesh=plsc.VectorSubcoreMesh(**_MESH),
        compiler_params=_SC_PARAMS,
        scratch_types=[
            pltpu.VMEM_SHARED((NPAD, CH), jnp.float32),
            pltpu.VMEM_SHARED((NPAD, CH), jnp.float32),
            pltpu.VMEM((NCH, EB), jnp.int32),
            pltpu.VMEM((NCH, EB), jnp.int32),
            pltpu.VMEM((rpt, CH), jnp.float32),
            pltpu.VMEM((rpt, CH), jnp.float32),
            pltpu.VMEM((EB, CH), jnp.float32),
            pltpu.VMEM((RSUB, CH), jnp.float32),
        ],
    )
    def prop_kernel(zt0_hbm, src_hbm, dst_hbm, u_hbm, ht_hbm, sout_hbm,
                    zt, agg, sidx, didx, ubuf, hbuf, rows, abuf):
        core = jax.lax.axis_index("c")
        tile = jax.lax.axis_index("s")
        r0 = tile * rpt
        h0 = core * NPAD + r0

        pltpu.sync_copy(src_hbm.at[pl.ds(tile * NCH, NCH)], sidx)
        pltpu.sync_copy(dst_hbm.at[pl.ds(tile * NCH, NCH)], didx)
        pltpu.sync_copy(u_hbm.at[pl.ds(r0, rpt)], ubuf)
        pltpu.sync_copy(ht_hbm.at[pl.ds(h0, rpt)], hbuf)
        for sub in range(nsub):
            ssp = pl.ds(r0 + sub * RSUB, RSUB)
            pltpu.sync_copy(zt0_hbm.at[pl.ds(h0 + sub * RSUB, RSUB)], abuf)
            pltpu.sync_copy(abuf, zt.at[ssp])
            pltpu.sync_copy(abuf, agg.at[ssp])
        plsc.subcore_barrier()

        def edge_pass():
            @pl.loop(0, NCH)
            def _(i):
                pltpu.sync_copy(zt.at[sidx.at[i]], rows)
                pltpu.sync_copy(rows, agg.at[didx.at[i]], add=True)

        @pl.loop(0, K - 1)
        def _(k):
            edge_pass()
            plsc.subcore_barrier()
            for sub in range(nsub):
                ssp = pl.ds(r0 + sub * RSUB, RSUB)
                pltpu.sync_copy(agg.at[ssp], abuf)

                @pl.loop(0, RSUB)
                def _(j):
                    jr = sub * RSUB + j
                    for cc in range(CH // 16):
                        csl = pl.ds(cc * 16, 16)
                        abuf[j, csl] = (ubuf[jr, csl] * abuf[j, csl]
                                        + hbuf[jr, csl])

                pltpu.sync_copy(abuf, zt.at[ssp])
                pltpu.sync_copy(abuf, agg.at[ssp])
            plsc.subcore_barrier()

        edge_pass()
        plsc.subcore_barrier()
        for sub in range(nsub):
            pltpu.sync_copy(agg.at[pl.ds(r0 + sub * RSUB, RSUB)], abuf)
            pltpu.sync_copy(abuf, sout_hbm.at[pl.ds(h0 + sub * RSUB, RSUB)])

    return prop_kernel(zt0f, src2d, dst2d, u32, htf)


def _tc_finish(sf, din64, ah64):
    n, c = din64.shape

    def body(s_ref, din_ref, ah_ref, o_ref):
        s = jnp.concatenate([s_ref[...][:n], s_ref[...][n:]], axis=1)
        z = din_ref[...] * s + ah_ref[...]
        m = jnp.max(z, axis=1, keepdims=True)
        lse = jnp.log(jnp.sum(jnp.exp(z - m), axis=1, keepdims=True)) + m
        o_ref[...] = z - lse

    return pl.pallas_call(
        body, out_shape=jax.ShapeDtypeStruct((n, c), jnp.float32),
    )(sf, din64, ah64)


def kernel(x, edge_index, W1, b1, W2, b2):
    n = x.shape[0]
    e = edge_index.shape[1]
    epad = NS * NCH * EB
    xp = jnp.pad(x, ((0, NPAD - n), (0, 0)))
    pad = jnp.full((epad - e,), NPAD - 1, jnp.int32)
    src2d = jnp.concatenate([edge_index[0], pad]).reshape(NS * NCH, EB)
    dst2d = jnp.concatenate([edge_index[1], pad]).reshape(NS * NCH, EB)
    deg16 = _sc_hist(dst2d)
    ztlo, zthi, u32, htlo, hthi, din64, ah64 = _tc_prep(xp, W1, b1, W2, b2, deg16)
    zt0f = jnp.concatenate([ztlo, zthi], axis=0)
    htf = jnp.concatenate([htlo, hthi], axis=0)
    sf = _sc_prop(zt0f, src2d, dst2d, u32, htf)
    return _tc_finish(sf, din64, ah64)[:n]


# speedup vs baseline: 26.2850x; 1.1464x over previous
import functools

import jax
import jax.numpy as jnp
from jax.experimental import pallas as pl
from jax.experimental.pallas import tpu as pltpu
from jax.experimental.pallas import tpu_sc as plsc

ALPHA = 0.1
K = 10
NS = 16
EB = 128
NCH = 160
RSUB = 128
NPAD = 10240
CH = 32

_SC_PARAMS = pltpu.CompilerParams(use_tc_tiling_on_sc=False)
_MESH = dict(core_axis_name="c", subcore_axis_name="s")


def _sc_hist(dst2d):
    hpt = NCH // 2
    rpt = NPAD // NS

    @functools.partial(
        pl.kernel,
        out_type=jax.ShapeDtypeStruct((2, NPAD, 16), jnp.float32),
        mesh=plsc.VectorSubcoreMesh(**_MESH),
        compiler_params=_SC_PARAMS,
        scratch_types=[
            pltpu.VMEM_SHARED((NPAD, 16), jnp.float32),
            pltpu.VMEM((rpt, 16), jnp.float32),
            pltpu.VMEM((EB, 16), jnp.float32),
            pltpu.VMEM((EB,), jnp.int32),
        ],
    )
    def hist_kernel(dst_hbm, out_hbm, acc, rowbuf, ones_b, didx):
        core = jax.lax.axis_index("c")
        tile = jax.lax.axis_index("s")
        zeros16 = jnp.zeros((16,), jnp.float32)
        ones16 = jnp.ones((16,), jnp.float32)

        @pl.loop(0, rpt)
        def _(j):
            rowbuf[j, :] = zeros16

        pltpu.sync_copy(rowbuf, acc.at[pl.ds(tile * rpt, rpt)])

        @pl.loop(0, EB)
        def _(j):
            ones_b[j, :] = ones16

        plsc.subcore_barrier()
        c0 = tile * NCH + core * hpt

        @pl.loop(0, hpt)
        def _(i):
            pltpu.sync_copy(dst_hbm.at[c0 + i], didx)
            pltpu.sync_copy(ones_b, acc.at[didx], add=True)

        plsc.subcore_barrier()
        pltpu.sync_copy(acc.at[pl.ds(tile * rpt, rpt)], rowbuf)
        pltpu.sync_copy(rowbuf, out_hbm.at[core].at[pl.ds(tile * rpt, rpt)])

    return hist_kernel(dst2d)


def _tc_prep(x, W1, b1, W2, b2, deg16):
    n = x.shape[0]
    c = W2.shape[1]

    blk = 1280
    grid = n // blk

    def body(x_ref, w1_ref, b1_ref, w2_ref, b2_ref, deg_ref,
             ztlo_ref, zthi_ref, u_ref, htlo_ref, hthi_ref, din_ref, ah_ref):
        h1 = jnp.maximum(
            jnp.dot(x_ref[...], w1_ref[...], preferred_element_type=jnp.float32)
            + b1_ref[...], 0.0)
        h = jnp.dot(h1, w2_ref[...], preferred_element_type=jnp.float32) + b2_ref[...]
        deg = deg_ref[...][0, :, 0:1] + deg_ref[...][1, :, 0:1] + 1.0
        dinv = jax.lax.rsqrt(deg)
        zt = dinv * h
        ztlo_ref[...] = zt[:, :CH]
        zthi_ref[...] = zt[:, CH:]
        htlo_ref[...] = ALPHA * zt[:, :CH]
        hthi_ref[...] = ALPHA * zt[:, CH:]
        u_ref[...] = jnp.broadcast_to((1.0 - ALPHA) * dinv * dinv, (blk, CH))
        din_ref[...] = jnp.broadcast_to((1.0 - ALPHA) * dinv, h.shape)
        ah_ref[...] = ALPHA * h

    f = jnp.float32
    row = lambda i: (i, 0)
    bs_h = pl.BlockSpec((blk, CH), row)
    bs_c = pl.BlockSpec((blk, c), row)
    return pl.pallas_call(
        body,
        grid=(grid,),
        in_specs=[
            pl.BlockSpec((blk, x.shape[1]), row),
            pl.BlockSpec(W1.shape, lambda i: (0, 0)),
            pl.BlockSpec((1, b1.shape[0]), lambda i: (0, 0)),
            pl.BlockSpec(W2.shape, lambda i: (0, 0)),
            pl.BlockSpec((1, b2.shape[0]), lambda i: (0, 0)),
            pl.BlockSpec((2, blk, 16), lambda i: (0, i, 0)),
        ],
        out_specs=[bs_h, bs_h, bs_h, bs_h, bs_h, bs_c, bs_c],
        out_shape=(
            jax.ShapeDtypeStruct((n, CH), f),
            jax.ShapeDtypeStruct((n, CH), f),
            jax.ShapeDtypeStruct((n, CH), f),
            jax.ShapeDtypeStruct((n, CH), f),
            jax.ShapeDtypeStruct((n, CH), f),
            jax.ShapeDtypeStruct((n, c), f),
            jax.ShapeDtypeStruct((n, c), f),
        ),
    )(x, W1, b1.reshape(1, -1), W2, b2.reshape(1, -1), deg16)


def _sc_prop(zt0f, src2d, dst2d, u32, htf):
    rpt = NPAD // NS
    nsub = rpt // RSUB

    @functools.partial(
        pl.kernel,
        out_type=jax.ShapeDtypeStruct((2 * NPAD, CH), jnp.float32),
        mesh=plsc.VectorSubcoreMesh(**_MESH),
        compiler_params=_SC_PARAMS,
        scratch_types=[
            pltpu.VMEM_SHARED((NPAD, CH), jnp.float32),
            pltpu.VMEM_SHARED((NPAD, CH), jnp.float32),
            pltpu.VMEM((NCH, EB), jnp.int32),
            pltpu.VMEM((NCH, EB), jnp.int32),
            pltpu.VMEM((RSUB, CH), jnp.float32),
            pltpu.VMEM((RSUB, CH), jnp.float32),
            pltpu.VMEM((EB, CH), jnp.float32),
            pltpu.VMEM((EB, CH), jnp.float32),
            pltpu.VMEM((RSUB, CH), jnp.float32),
            pltpu.SemaphoreType.DMA,
            pltpu.SemaphoreType.DMA,
            pltpu.SemaphoreType.DMA,
            pltpu.SemaphoreType.DMA,
        ],
    )
    def prop_kernel(zt0_hbm, src_hbm, dst_hbm, u_hbm, ht_hbm, sout_hbm,
                    zt, agg, sidx, didx, ubuf, hbuf, rows_a, rows_b, abuf,
                    gsa, gsb, ssa, ssb):
        core = jax.lax.axis_index("c")
        tile = jax.lax.axis_index("s")
        r0 = tile * rpt
        h0 = core * NPAD + r0

        pltpu.sync_copy(src_hbm.at[pl.ds(tile * NCH, NCH)], sidx)
        pltpu.sync_copy(dst_hbm.at[pl.ds(tile * NCH, NCH)], didx)
        for sub in range(nsub):
            ssp = pl.ds(r0 + sub * RSUB, RSUB)
            pltpu.sync_copy(zt0_hbm.at[pl.ds(h0 + sub * RSUB, RSUB)], abuf)
            pltpu.sync_copy(abuf, zt.at[ssp])
            pltpu.sync_copy(abuf, agg.at[ssp])
        plsc.subcore_barrier()

        def g_start(i, buf, sem):
            pltpu.async_copy(zt.at[sidx.at[i]], buf, sem)

        def g_wait(buf, sem):
            pltpu.make_async_copy(zt.at[sidx.at[0]], buf, sem).wait()

        def s_start(i, buf, sem):
            pltpu.async_copy(buf, agg.at[didx.at[i]], sem, add=True)

        def s_wait(buf, sem):
            pltpu.make_async_copy(buf, agg.at[didx.at[0]], sem).wait()

        npairs = NCH // 2

        def edge_pass():
            g_start(0, rows_a, gsa)

            @pl.loop(0, npairs)
            def _(p):
                i0 = 2 * p
                g_wait(rows_a, gsa)

                @pl.when(p > 0)
                def _():
                    s_wait(rows_b, ssb)

                g_start(i0 + 1, rows_b, gsb)
                s_start(i0, rows_a, ssa)
                g_wait(rows_b, gsb)
                s_wait(rows_a, ssa)

                @pl.when(p < npairs - 1)
                def _():
                    g_start(i0 + 2, rows_a, gsa)

                s_start(i0 + 1, rows_b, ssb)

            s_wait(rows_b, ssb)

        @pl.loop(0, K - 1)
        def _(k):
            edge_pass()
            plsc.subcore_barrier()
            for sub in range(nsub):
                ssp = pl.ds(r0 + sub * RSUB, RSUB)
                pltpu.sync_copy(agg.at[ssp], abuf)
                pltpu.sync_copy(u_hbm.at[pl.ds(r0 + sub * RSUB, RSUB)], ubuf)
                pltpu.sync_copy(ht_hbm.at[pl.ds(h0 + sub * RSUB, RSUB)], hbuf)

                @pl.loop(0, RSUB)
                def _(j):
                    for cc in range(CH // 16):
                        csl = pl.ds(cc * 16, 16)
                        abuf[j, csl] = (ubuf[j, csl] * abuf[j, csl]
                                        + hbuf[j, csl])

                pltpu.sync_copy(abuf, zt.at[ssp])
                pltpu.sync_copy(abuf, agg.at[ssp])
            plsc.subcore_barrier()

        edge_pass()
        plsc.subcore_barrier()
        for sub in range(nsub):
            pltpu.sync_copy(agg.at[pl.ds(r0 + sub * RSUB, RSUB)], abuf)
            pltpu.sync_copy(abuf, sout_hbm.at[pl.ds(h0 + sub * RSUB, RSUB)])

    return prop_kernel(zt0f, src2d, dst2d, u32, htf)


def _tc_finish(sf, din64, ah64):
    n, c = din64.shape

    def body(s_ref, din_ref, ah_ref, o_ref):
        s = jnp.concatenate([s_ref[...][:n], s_ref[...][n:]], axis=1)
        z = din_ref[...] * s + ah_ref[...]
        m = jnp.max(z, axis=1, keepdims=True)
        lse = jnp.log(jnp.sum(jnp.exp(z - m), axis=1, keepdims=True)) + m
        o_ref[...] = z - lse

    return pl.pallas_call(
        body, out_shape=jax.ShapeDtypeStruct((n, c), jnp.float32),
    )(sf, din64, ah64)


def kernel(x, edge_index, W1, b1, W2, b2):
    n = x.shape[0]
    e = edge_index.shape[1]
    epad = NS * NCH * EB
    xp = jnp.pad(x, ((0, NPAD - n), (0, 0)))
    pad = jnp.full((epad - e,), NPAD - 1, jnp.int32)
    src2d = jnp.concatenate([edge_index[0], pad]).reshape(NS * NCH, EB)
    dst2d = jnp.concatenate([edge_index[1], pad]).reshape(NS * NCH, EB)
    deg16 = _sc_hist(dst2d)
    ztlo, zthi, u32, htlo, hthi, din64, ah64 = _tc_prep(xp, W1, b1, W2, b2, deg16)
    zt0f = jnp.concatenate([ztlo, zthi], axis=0)
    htf = jnp.concatenate([htlo, hthi], axis=0)
    sf = _sc_prop(zt0f, src2d, dst2d, u32, htf)
    return _tc_finish(sf, din64, ah64)[:n]
